# Initial kernel scaffold; baseline (speedup 1.0000x reference)
import jax
import jax.numpy as jnp
from jax.experimental import pallas as pl


def kernel(features, W_conv, b_conv, W_logits, b_logits, W_deltas, b_deltas):
    raise NotImplementedError("write your pallas kernel here")


# speedup vs baseline: 1.0187x; 1.0187x over previous
import functools

import jax
import jax.numpy as jnp
import numpy as np
from jax.experimental import pallas as pl
from jax.experimental.pallas import tpu as pltpu

H = 50
W = 50
C = 256
STRIDE = 16.0
IMG = 800.0
SIZES = (32.0, 64.0, 128.0, 256.0, 512.0)
RATIOS = (0.5, 1.0, 2.0)
A = 15
PRE_NMS = 2000
POST_NMS = 1000
NMS_THRESH = 0.7
LOGMAX = float(np.log(1000.0 / 16.0))

HP = H + 2
WP = W + 2
P = H * W
AP = 16


def _anchor_wh():
    ws, hs = [], []
    for s in SIZES:
        area = s * s
        for r in RATIOS:
            bw = float(np.sqrt(area / r))
            bh = bw * r
            ws.append(bw)
            hs.append(bh)
    wa = np.zeros((1, AP), np.float32)
    ha = np.zeros((1, AP), np.float32)
    wa[0, :A] = ws
    ha[0, :A] = hs
    return wa, ha


def _dense_body(xpad_ref, w9_ref, bconv_ref, whead_ref, bhead_ref,
                wa_ref, ha_ref,
                score_ref, x1_ref, y1_ref, x2_ref, y2_ref):
    X = xpad_ref[...]
    acc = jnp.zeros((H, W, C), jnp.float32)
    for k in range(9):
        dy, dx = k // 3, k % 3
        Y = jnp.dot(X, w9_ref[k], preferred_element_type=jnp.float32)
        Y = Y.reshape(HP, WP, C)
        acc = acc + Y[dy:dy + H, dx:dx + W, :]
    t = jnp.maximum(acc.reshape(P, C) + bconv_ref[...], 0.0)
    h = jnp.dot(t, whead_ref[...], preferred_element_type=jnp.float32)
    h = h + bhead_ref[...]
    logits = h[:, 0:AP]
    dxv = h[:, AP:2 * AP]
    dyv = h[:, 2 * AP:3 * AP]
    dwv = h[:, 3 * AP:4 * AP]
    dhv = h[:, 4 * AP:5 * AP]

    pidx = jax.lax.broadcasted_iota(jnp.int32, (P, AP), 0)
    px = (pidx % W).astype(jnp.float32) * STRIDE
    py = (pidx // W).astype(jnp.float32) * STRIDE

    wa = wa_ref[...]
    ha = ha_ref[...]
    cx = dxv * wa + px
    cy = dyv * ha + py
    pw = jnp.exp(jnp.minimum(dwv, LOGMAX)) * wa
    ph = jnp.exp(jnp.minimum(dhv, LOGMAX)) * ha
    score_ref[...] = logits[:, :A]
    x1_ref[...] = jnp.clip(cx - 0.5 * pw, 0.0, IMG)[:, :A]
    y1_ref[...] = jnp.clip(cy - 0.5 * ph, 0.0, IMG)[:, :A]
    x2_ref[...] = jnp.clip(cx + 0.5 * pw, 0.0, IMG)[:, :A]
    y2_ref[...] = jnp.clip(cy + 0.5 * ph, 0.0, IMG)[:, :A]


@functools.partial(jax.jit, static_argnames=("interpret",))
def _dense_stage(features, W_conv, b_conv, W_logits, b_logits, W_deltas,
                 b_deltas, interpret=False):
    xpad = jnp.pad(features[0], ((1, 1), (1, 1), (0, 0)))
    xpad = xpad.reshape(HP * WP, C)
    w9 = W_conv.reshape(9, C, C)
    bconv = b_conv.reshape(1, C)
    wl = W_logits[0, 0]
    wd = W_deltas[0, 0]
    cols = [jnp.pad(wl, ((0, 0), (0, AP - A)))]
    bcols = [jnp.pad(b_logits, (0, AP - A))]
    for k in range(4):
        cols.append(jnp.pad(wd[:, k::4], ((0, 0), (0, AP - A))))
        bcols.append(jnp.pad(b_deltas[k::4], (0, AP - A)))
    whead = jnp.concatenate(cols, axis=1)
    bhead = jnp.concatenate(bcols).reshape(1, 5 * AP)
    wa_np, ha_np = _anchor_wh()
    out_sds = [jax.ShapeDtypeStruct((P, A), jnp.float32)] * 5
    outs = pl.pallas_call(
        _dense_body,
        out_shape=out_sds,
        interpret=interpret,
    )(xpad, w9, bconv, whead, bhead, jnp.asarray(wa_np), jnp.asarray(ha_np))
    score, x1, y1, x2, y2 = outs
    scores = score.reshape(-1)
    boxes = jnp.stack([x1, y1, x2, y2], axis=-1).reshape(-1, 4)
    return scores, boxes


def _nms_keep(boxes):
    x1, y1, x2, y2 = boxes[:, 0], boxes[:, 1], boxes[:, 2], boxes[:, 3]
    areas = (x2 - x1) * (y2 - y1)
    ix1 = jnp.maximum(x1[:, None], x1[None, :])
    iy1 = jnp.maximum(y1[:, None], y1[None, :])
    ix2 = jnp.minimum(x2[:, None], x2[None, :])
    iy2 = jnp.minimum(y2[:, None], y2[None, :])
    inter = jnp.clip(ix2 - ix1, 0.0) * jnp.clip(iy2 - iy1, 0.0)
    iou = inter / (areas[:, None] + areas[None, :] - inter + 1e-9)
    n = boxes.shape[0]
    ar = jnp.arange(n)

    def body(i, keep):
        cur = keep[i]
        sup = (iou[i] > NMS_THRESH) & (ar > i)
        return jnp.where(cur, keep & (~sup), keep)

    return jax.lax.fori_loop(0, n, body, jnp.ones((n,), bool))


def kernel(features, W_conv, b_conv, W_logits, b_logits, W_deltas, b_deltas):
    scores, boxes = _dense_stage(features, W_conv, b_conv, W_logits,
                                 b_logits, W_deltas, b_deltas)
    top_scores, idx = jax.lax.top_k(scores, PRE_NMS)
    top_boxes = boxes[idx]
    keep = _nms_keep(top_boxes)
    kept_scores = jnp.where(keep, top_scores, -1e9)
    final_scores, fidx = jax.lax.top_k(kept_scores, POST_NMS)
    final_boxes = top_boxes[fidx]
    return jnp.concatenate([final_boxes, final_scores[:, None]], axis=1)


# speedup vs baseline: 11.8903x; 11.6720x over previous
import functools

import jax
import jax.numpy as jnp
import numpy as np
from jax.experimental import pallas as pl
from jax.experimental.pallas import tpu as pltpu

H = 50
W = 50
C = 256
STRIDE = 16.0
IMG = 800.0
SIZES = (32.0, 64.0, 128.0, 256.0, 512.0)
RATIOS = (0.5, 1.0, 2.0)
A = 15
PRE_NMS = 2000
POST_NMS = 1000
NMS_THRESH = 0.7
LOGMAX = float(np.log(1000.0 / 16.0))

HP = H + 2
WP = W + 2
P = H * W
AP = 16


def _anchor_wh():
    ws, hs = [], []
    for s in SIZES:
        area = s * s
        for r in RATIOS:
            bw = float(np.sqrt(area / r))
            bh = bw * r
            ws.append(bw)
            hs.append(bh)
    wa = np.zeros((1, AP), np.float32)
    ha = np.zeros((1, AP), np.float32)
    wa[0, :A] = ws
    ha[0, :A] = hs
    return wa, ha


def _dense_body(xpad_ref, w9_ref, bconv_ref, whead_ref, bhead_ref,
                wa_ref, ha_ref,
                score_ref, x1_ref, y1_ref, x2_ref, y2_ref):
    X = xpad_ref[...]
    acc = jnp.zeros((H, W, C), jnp.float32)
    for k in range(9):
        dy, dx = k // 3, k % 3
        Y = jnp.dot(X, w9_ref[k], preferred_element_type=jnp.float32)
        Y = Y.reshape(HP, WP, C)
        acc = acc + Y[dy:dy + H, dx:dx + W, :]
    t = jnp.maximum(acc.reshape(P, C) + bconv_ref[...], 0.0)
    h = jnp.dot(t, whead_ref[...], preferred_element_type=jnp.float32)
    h = h + bhead_ref[...]
    logits = h[:, 0:AP]
    dxv = h[:, AP:2 * AP]
    dyv = h[:, 2 * AP:3 * AP]
    dwv = h[:, 3 * AP:4 * AP]
    dhv = h[:, 4 * AP:5 * AP]

    pidx = jax.lax.broadcasted_iota(jnp.int32, (P, AP), 0)
    px = (pidx % W).astype(jnp.float32) * STRIDE
    py = (pidx // W).astype(jnp.float32) * STRIDE

    wa = wa_ref[...]
    ha = ha_ref[...]
    cx = dxv * wa + px
    cy = dyv * ha + py
    pw = jnp.exp(jnp.minimum(dwv, LOGMAX)) * wa
    ph = jnp.exp(jnp.minimum(dhv, LOGMAX)) * ha
    score_ref[...] = logits[:, :A]
    x1_ref[...] = jnp.clip(cx - 0.5 * pw, 0.0, IMG)[:, :A]
    y1_ref[...] = jnp.clip(cy - 0.5 * ph, 0.0, IMG)[:, :A]
    x2_ref[...] = jnp.clip(cx + 0.5 * pw, 0.0, IMG)[:, :A]
    y2_ref[...] = jnp.clip(cy + 0.5 * ph, 0.0, IMG)[:, :A]


@functools.partial(jax.jit, static_argnames=("interpret",))
def _dense_stage(features, W_conv, b_conv, W_logits, b_logits, W_deltas,
                 b_deltas, interpret=False):
    xpad = jnp.pad(features[0], ((1, 1), (1, 1), (0, 0)))
    xpad = xpad.reshape(HP * WP, C)
    w9 = W_conv.reshape(9, C, C)
    bconv = b_conv.reshape(1, C)
    wl = W_logits[0, 0]
    wd = W_deltas[0, 0]
    cols = [jnp.pad(wl, ((0, 0), (0, AP - A)))]
    bcols = [jnp.pad(b_logits, (0, AP - A))]
    for k in range(4):
        cols.append(jnp.pad(wd[:, k::4], ((0, 0), (0, AP - A))))
        bcols.append(jnp.pad(b_deltas[k::4], (0, AP - A)))
    whead = jnp.concatenate(cols, axis=1)
    bhead = jnp.concatenate(bcols).reshape(1, 5 * AP)
    wa_np, ha_np = _anchor_wh()
    out_sds = [jax.ShapeDtypeStruct((P, A), jnp.float32)] * 5
    outs = pl.pallas_call(
        _dense_body,
        out_shape=out_sds,
        interpret=interpret,
    )(xpad, w9, bconv, whead, bhead, jnp.asarray(wa_np), jnp.asarray(ha_np))
    score, x1, y1, x2, y2 = outs
    scores = score.reshape(-1)
    boxes = jnp.stack([x1, y1, x2, y2], axis=-1).reshape(-1, 4)
    return scores, boxes


N2 = 2048


def _nms_body(data8_ref, bxr_ref, out_ref, sup_ref):
    x1r = bxr_ref[0:1, :]
    y1r = bxr_ref[1:2, :]
    x2r = bxr_ref[2:3, :]
    y2r = bxr_ref[3:4, :]
    area_r = (x2r - x1r) * (y2r - y1r)
    for b in range(N2 // 128):
        x1c = data8_ref[b * 128:(b + 1) * 128, 0:1]
        y1c = data8_ref[b * 128:(b + 1) * 128, 1:2]
        x2c = data8_ref[b * 128:(b + 1) * 128, 2:3]
        y2c = data8_ref[b * 128:(b + 1) * 128, 3:4]
        ix1 = jnp.maximum(x1c, x1r)
        iy1 = jnp.maximum(y1c, y1r)
        ix2 = jnp.minimum(x2c, x2r)
        iy2 = jnp.minimum(y2c, y2r)
        inter = jnp.clip(ix2 - ix1, 0.0) * jnp.clip(iy2 - iy1, 0.0)
        area_c = (x2c - x1c) * (y2c - y1c)
        iou = inter / (area_c + area_r - inter + 1e-9)
        irow = jax.lax.broadcasted_iota(jnp.int32, (128, N2), 0) + b * 128
        jrow = jax.lax.broadcasted_iota(jnp.int32, (128, N2), 1)
        sup_ref[b * 128:(b + 1) * 128, :] = jnp.where(
            (iou > NMS_THRESH) & (jrow > irow), 1.0, 0.0)

    iota_lane = jax.lax.broadcasted_iota(jnp.int32, (1, N2), 1)

    def body(i, keep):
        row = sup_ref[pl.ds(i, 1), :]
        cur = jnp.sum(jnp.where(iota_lane == i, keep, 0.0))
        return keep * (1.0 - cur * row)

    keep = jax.lax.fori_loop(0, PRE_NMS, body,
                             jnp.ones((1, N2), jnp.float32))

    valid = iota_lane < PRE_NMS
    keepv = jnp.where(valid, keep, 0.0)
    suppv = jnp.where(valid, 1.0 - keep, 0.0)
    K = jnp.sum(keepv)
    kp = keepv.reshape(16, 128)
    sp = suppv.reshape(16, 128)
    l0 = jax.lax.broadcasted_iota(jnp.int32, (128, 128), 0)
    l1 = jax.lax.broadcasted_iota(jnp.int32, (128, 128), 1)
    slt = jnp.where(l0 < l1, 1.0, 0.0)
    r0 = jax.lax.broadcasted_iota(jnp.int32, (16, 16), 0)
    r1 = jax.lax.broadcasted_iota(jnp.int32, (16, 16), 1)
    mrow = jnp.where(r1 < r0, 1.0, 0.0)
    ke = jnp.dot(kp, slt, preferred_element_type=jnp.float32) + jnp.dot(
        mrow, jnp.sum(kp, axis=1, keepdims=True),
        preferred_element_type=jnp.float32)
    se = jnp.dot(sp, slt, preferred_element_type=jnp.float32) + jnp.dot(
        mrow, jnp.sum(sp, axis=1, keepdims=True),
        preferred_element_type=jnp.float32)
    dest = jnp.where(kp > 0.5, ke, K + se)
    g16 = jax.lax.broadcasted_iota(jnp.int32, (16, 128), 0) * 128 + \
        jax.lax.broadcasted_iota(jnp.int32, (16, 128), 1)
    dest = jnp.where(g16 < PRE_NMS, dest, float(N2 - 1))
    destrow = dest.reshape(1, N2)
    dcol = jax.lax.broadcasted_iota(jnp.int32, (N2, 1), 0).astype(jnp.float32)
    onehot_t = jnp.where(dcol == destrow, 1.0, 0.0)
    out = jnp.dot(onehot_t, data8_ref[...], preferred_element_type=jnp.float32)
    lane8 = jax.lax.broadcasted_iota(jnp.int32, (N2, 8), 1)
    out = jnp.where((lane8 == 4) & (dcol >= K), -1e9, out)
    out_ref[...] = out


@functools.partial(jax.jit, static_argnames=("interpret",))
def _nms_stage(top_scores, top_boxes, interpret=False):
    data8 = jnp.concatenate([
        top_boxes, top_scores[:, None],
        jnp.zeros((PRE_NMS, 3), jnp.float32)], axis=1)
    data8 = jnp.pad(data8, ((0, N2 - PRE_NMS), (0, 0)))
    bx_row = data8[:, :4].T
    out = pl.pallas_call(
        _nms_body,
        out_shape=jax.ShapeDtypeStruct((N2, 8), jnp.float32),
        scratch_shapes=[pltpu.VMEM((N2, N2), jnp.float32)],
        interpret=interpret,
    )(data8, bx_row)
    return out[:POST_NMS, :5]


def kernel(features, W_conv, b_conv, W_logits, b_logits, W_deltas, b_deltas):
    scores, boxes = _dense_stage(features, W_conv, b_conv, W_logits,
                                 b_logits, W_deltas, b_deltas)
    top_scores, idx = jax.lax.top_k(scores, PRE_NMS)
    top_boxes = boxes[idx]
    return _nms_stage(top_scores, top_boxes)


# speedup vs baseline: 12.7009x; 1.0682x over previous
import functools

import jax
import jax.numpy as jnp
import numpy as np
from jax.experimental import pallas as pl
from jax.experimental.pallas import tpu as pltpu

H = 50
W = 50
C = 256
STRIDE = 16.0
IMG = 800.0
SIZES = (32.0, 64.0, 128.0, 256.0, 512.0)
RATIOS = (0.5, 1.0, 2.0)
A = 15
PRE_NMS = 2000
POST_NMS = 1000
NMS_THRESH = 0.7
LOGMAX = float(np.log(1000.0 / 16.0))

HP = H + 2
WP = W + 2
P = H * W
AP = 16


def _anchor_wh():
    ws, hs = [], []
    for s in SIZES:
        area = s * s
        for r in RATIOS:
            bw = float(np.sqrt(area / r))
            bh = bw * r
            ws.append(bw)
            hs.append(bh)
    wa = np.zeros((1, AP), np.float32)
    ha = np.zeros((1, AP), np.float32)
    wa[0, :A] = ws
    ha[0, :A] = hs
    return wa, ha


def _dense_body(xpad_ref, w9_ref, bconv_ref, whead_ref, bhead_ref,
                wa_ref, ha_ref,
                score_ref, x1_ref, y1_ref, x2_ref, y2_ref):
    X = xpad_ref[...]
    acc = jnp.zeros((H, W, C), jnp.float32)
    for k in range(9):
        dy, dx = k // 3, k % 3
        Y = jnp.dot(X, w9_ref[k], preferred_element_type=jnp.float32)
        Y = Y.reshape(HP, WP, C)
        acc = acc + Y[dy:dy + H, dx:dx + W, :]
    t = jnp.maximum(acc.reshape(P, C) + bconv_ref[...], 0.0)
    h = jnp.dot(t, whead_ref[...], preferred_element_type=jnp.float32)
    h = h + bhead_ref[...]
    logits = h[:, 0:AP]
    dxv = h[:, AP:2 * AP]
    dyv = h[:, 2 * AP:3 * AP]
    dwv = h[:, 3 * AP:4 * AP]
    dhv = h[:, 4 * AP:5 * AP]

    pidx = jax.lax.broadcasted_iota(jnp.int32, (P, AP), 0)
    px = (pidx % W).astype(jnp.float32) * STRIDE
    py = (pidx // W).astype(jnp.float32) * STRIDE

    wa = wa_ref[...]
    ha = ha_ref[...]
    cx = dxv * wa + px
    cy = dyv * ha + py
    pw = jnp.exp(jnp.minimum(dwv, LOGMAX)) * wa
    ph = jnp.exp(jnp.minimum(dhv, LOGMAX)) * ha
    score_ref[...] = logits[:, :A]
    x1_ref[...] = jnp.clip(cx - 0.5 * pw, 0.0, IMG)[:, :A]
    y1_ref[...] = jnp.clip(cy - 0.5 * ph, 0.0, IMG)[:, :A]
    x2_ref[...] = jnp.clip(cx + 0.5 * pw, 0.0, IMG)[:, :A]
    y2_ref[...] = jnp.clip(cy + 0.5 * ph, 0.0, IMG)[:, :A]


@functools.partial(jax.jit, static_argnames=("interpret",))
def _dense_stage(features, W_conv, b_conv, W_logits, b_logits, W_deltas,
                 b_deltas, interpret=False):
    xpad = jnp.pad(features[0], ((1, 1), (1, 1), (0, 0)))
    xpad = xpad.reshape(HP * WP, C)
    w9 = W_conv.reshape(9, C, C)
    bconv = b_conv.reshape(1, C)
    wl = W_logits[0, 0]
    wd = W_deltas[0, 0]
    cols = [jnp.pad(wl, ((0, 0), (0, AP - A)))]
    bcols = [jnp.pad(b_logits, (0, AP - A))]
    for k in range(4):
        cols.append(jnp.pad(wd[:, k::4], ((0, 0), (0, AP - A))))
        bcols.append(jnp.pad(b_deltas[k::4], (0, AP - A)))
    whead = jnp.concatenate(cols, axis=1)
    bhead = jnp.concatenate(bcols).reshape(1, 5 * AP)
    wa_np, ha_np = _anchor_wh()
    out_sds = [jax.ShapeDtypeStruct((P, A), jnp.float32)] * 5
    outs = pl.pallas_call(
        _dense_body,
        out_shape=out_sds,
        interpret=interpret,
    )(xpad, w9, bconv, whead, bhead, jnp.asarray(wa_np), jnp.asarray(ha_np))
    score, x1, y1, x2, y2 = outs
    scores = score.reshape(-1)
    boxes = jnp.stack([x1, y1, x2, y2], axis=-1).reshape(-1, 4)
    return scores, boxes


N2 = 2048


def _nms_body(data8_ref, bxr_ref, out_ref, sup_ref):
    x1r = bxr_ref[0:1, :]
    y1r = bxr_ref[1:2, :]
    x2r = bxr_ref[2:3, :]
    y2r = bxr_ref[3:4, :]
    area_r = (x2r - x1r) * (y2r - y1r)
    for b in range(N2 // 128):
        x1c = data8_ref[b * 128:(b + 1) * 128, 0:1]
        y1c = data8_ref[b * 128:(b + 1) * 128, 1:2]
        x2c = data8_ref[b * 128:(b + 1) * 128, 2:3]
        y2c = data8_ref[b * 128:(b + 1) * 128, 3:4]
        ix1 = jnp.maximum(x1c, x1r)
        iy1 = jnp.maximum(y1c, y1r)
        ix2 = jnp.minimum(x2c, x2r)
        iy2 = jnp.minimum(y2c, y2r)
        inter = jnp.clip(ix2 - ix1, 0.0) * jnp.clip(iy2 - iy1, 0.0)
        area_c = (x2c - x1c) * (y2c - y1c)
        iou = inter / (area_c + area_r - inter + 1e-9)
        irow = jax.lax.broadcasted_iota(jnp.int32, (128, N2), 0) + b * 128
        jrow = jax.lax.broadcasted_iota(jnp.int32, (128, N2), 1)
        sup_ref[b * 128:(b + 1) * 128, :] = jnp.where(
            (iou > NMS_THRESH) & (jrow > irow), 1.0, 0.0)

    iota_lane = jax.lax.broadcasted_iota(jnp.int32, (1, N2), 1)
    iota128 = jax.lax.broadcasted_iota(jnp.int32, (1, 128), 1)
    keep = jnp.ones((1, N2), jnp.float32)
    for b in range(N2 // 128):
        lo = b * 128

        def blk_body(i, bk, lo=lo):
            row = sup_ref[pl.ds(lo + i, 1), :][:, lo:lo + 128]
            cur = jnp.sum(jnp.where(iota128 == i, bk, 0.0))
            return bk * (1.0 - cur * row)

        blkkeep = jax.lax.fori_loop(0, 128, blk_body, keep[:, lo:lo + 128])
        nsup = jnp.dot(blkkeep, sup_ref[lo:lo + 128, :],
                       preferred_element_type=jnp.float32)
        keep = keep * jnp.where(nsup > 0.0, 0.0, 1.0)

    valid = iota_lane < PRE_NMS
    keepv = jnp.where(valid, keep, 0.0)
    suppv = jnp.where(valid, 1.0 - keep, 0.0)
    K = jnp.sum(keepv)
    kp = keepv.reshape(16, 128)
    sp = suppv.reshape(16, 128)
    l0 = jax.lax.broadcasted_iota(jnp.int32, (128, 128), 0)
    l1 = jax.lax.broadcasted_iota(jnp.int32, (128, 128), 1)
    slt = jnp.where(l0 < l1, 1.0, 0.0)
    r0 = jax.lax.broadcasted_iota(jnp.int32, (16, 16), 0)
    r1 = jax.lax.broadcasted_iota(jnp.int32, (16, 16), 1)
    mrow = jnp.where(r1 < r0, 1.0, 0.0)
    ke = jnp.dot(kp, slt, preferred_element_type=jnp.float32) + jnp.dot(
        mrow, jnp.sum(kp, axis=1, keepdims=True),
        preferred_element_type=jnp.float32)
    se = jnp.dot(sp, slt, preferred_element_type=jnp.float32) + jnp.dot(
        mrow, jnp.sum(sp, axis=1, keepdims=True),
        preferred_element_type=jnp.float32)
    dest = jnp.where(kp > 0.5, ke, K + se)
    g16 = jax.lax.broadcasted_iota(jnp.int32, (16, 128), 0) * 128 + \
        jax.lax.broadcasted_iota(jnp.int32, (16, 128), 1)
    dest = jnp.where(g16 < PRE_NMS, dest, float(N2 - 1))
    destrow = dest.reshape(1, N2)
    dcol = jax.lax.broadcasted_iota(jnp.int32, (N2, 1), 0).astype(jnp.float32)
    onehot_t = jnp.where(dcol == destrow, 1.0, 0.0)
    out = jnp.dot(onehot_t, data8_ref[...], preferred_element_type=jnp.float32)
    lane8 = jax.lax.broadcasted_iota(jnp.int32, (N2, 8), 1)
    out = jnp.where((lane8 == 4) & (dcol >= K), -1e9, out)
    out_ref[...] = out


@functools.partial(jax.jit, static_argnames=("interpret",))
def _nms_stage(top_scores, top_boxes, interpret=False):
    data8 = jnp.concatenate([
        top_boxes, top_scores[:, None],
        jnp.zeros((PRE_NMS, 3), jnp.float32)], axis=1)
    data8 = jnp.pad(data8, ((0, N2 - PRE_NMS), (0, 0)))
    bx_row = data8[:, :4].T
    out = pl.pallas_call(
        _nms_body,
        out_shape=jax.ShapeDtypeStruct((N2, 8), jnp.float32),
        scratch_shapes=[pltpu.VMEM((N2, N2), jnp.float32)],
        interpret=interpret,
    )(data8, bx_row)
    return out[:POST_NMS, :5]


def kernel(features, W_conv, b_conv, W_logits, b_logits, W_deltas, b_deltas):
    scores, boxes = _dense_stage(features, W_conv, b_conv, W_logits,
                                 b_logits, W_deltas, b_deltas)
    top_scores, idx = jax.lax.top_k(scores, PRE_NMS)
    top_boxes = boxes[idx]
    return _nms_stage(top_scores, top_boxes)


# speedup vs baseline: 12.7015x; 1.0000x over previous
import functools

import jax
import jax.numpy as jnp
import numpy as np
from jax.experimental import pallas as pl
from jax.experimental.pallas import tpu as pltpu

H = 50
W = 50
C = 256
STRIDE = 16.0
IMG = 800.0
SIZES = (32.0, 64.0, 128.0, 256.0, 512.0)
RATIOS = (0.5, 1.0, 2.0)
A = 15
PRE_NMS = 2000
POST_NMS = 1000
NMS_THRESH = 0.7
LOGMAX = float(np.log(1000.0 / 16.0))

HP = H + 2
WP = W + 2
P = H * W
AP = 16


def _anchor_wh():
    ws, hs = [], []
    for s in SIZES:
        area = s * s
        for r in RATIOS:
            bw = float(np.sqrt(area / r))
            bh = bw * r
            ws.append(bw)
            hs.append(bh)
    wa = np.zeros((1, AP), np.float32)
    ha = np.zeros((1, AP), np.float32)
    wa[0, :A] = ws
    ha[0, :A] = hs
    return wa, ha


def _dense_body(xpad_ref, w9_ref, bconv_ref, whead_ref, bhead_ref,
                wa_ref, ha_ref,
                score_ref, x1_ref, y1_ref, x2_ref, y2_ref):
    X = xpad_ref[...]
    acc = jnp.zeros((H, W, C), jnp.float32)
    for k in range(9):
        dy, dx = k // 3, k % 3
        Y = jnp.dot(X, w9_ref[k], preferred_element_type=jnp.float32)
        Y = Y.reshape(HP, WP, C)
        acc = acc + Y[dy:dy + H, dx:dx + W, :]
    t = jnp.maximum(acc.reshape(P, C) + bconv_ref[...], 0.0)
    h = jnp.dot(t, whead_ref[...], preferred_element_type=jnp.float32)
    h = h + bhead_ref[...]
    logits = h[:, 0:AP]
    dxv = h[:, AP:2 * AP]
    dyv = h[:, 2 * AP:3 * AP]
    dwv = h[:, 3 * AP:4 * AP]
    dhv = h[:, 4 * AP:5 * AP]

    pidx = jax.lax.broadcasted_iota(jnp.int32, (P, AP), 0)
    px = (pidx % W).astype(jnp.float32) * STRIDE
    py = (pidx // W).astype(jnp.float32) * STRIDE

    wa = wa_ref[...]
    ha = ha_ref[...]
    cx = dxv * wa + px
    cy = dyv * ha + py
    pw = jnp.exp(jnp.minimum(dwv, LOGMAX)) * wa
    ph = jnp.exp(jnp.minimum(dhv, LOGMAX)) * ha
    score_ref[...] = logits[:, :A]
    x1_ref[...] = jnp.clip(cx - 0.5 * pw, 0.0, IMG)[:, :A]
    y1_ref[...] = jnp.clip(cy - 0.5 * ph, 0.0, IMG)[:, :A]
    x2_ref[...] = jnp.clip(cx + 0.5 * pw, 0.0, IMG)[:, :A]
    y2_ref[...] = jnp.clip(cy + 0.5 * ph, 0.0, IMG)[:, :A]


@functools.partial(jax.jit, static_argnames=("interpret",))
def _dense_stage(features, W_conv, b_conv, W_logits, b_logits, W_deltas,
                 b_deltas, interpret=False):
    xpad = jnp.pad(features[0], ((1, 1), (1, 1), (0, 0)))
    xpad = xpad.reshape(HP * WP, C)
    w9 = W_conv.reshape(9, C, C)
    bconv = b_conv.reshape(1, C)
    wl = W_logits[0, 0]
    wd = W_deltas[0, 0]
    cols = [jnp.pad(wl, ((0, 0), (0, AP - A)))]
    bcols = [jnp.pad(b_logits, (0, AP - A))]
    for k in range(4):
        cols.append(jnp.pad(wd[:, k::4], ((0, 0), (0, AP - A))))
        bcols.append(jnp.pad(b_deltas[k::4], (0, AP - A)))
    whead = jnp.concatenate(cols, axis=1)
    bhead = jnp.concatenate(bcols).reshape(1, 5 * AP)
    wa_np, ha_np = _anchor_wh()
    out_sds = [jax.ShapeDtypeStruct((P, A), jnp.float32)] * 5
    outs = pl.pallas_call(
        _dense_body,
        out_shape=out_sds,
        interpret=interpret,
    )(xpad, w9, bconv, whead, bhead, jnp.asarray(wa_np), jnp.asarray(ha_np))
    score, x1, y1, x2, y2 = outs
    scores = score.reshape(-1)
    boxes = jnp.stack([x1, y1, x2, y2], axis=-1).reshape(-1, 4)
    return scores, boxes


N2 = 2048


def _nms_body(data8_ref, bxr_ref, out_ref, sup_ref):
    x1r = bxr_ref[0:1, :]
    y1r = bxr_ref[1:2, :]
    x2r = bxr_ref[2:3, :]
    y2r = bxr_ref[3:4, :]
    area_r = (x2r - x1r) * (y2r - y1r)
    for b in range(N2 // 128):
        x1c = data8_ref[b * 128:(b + 1) * 128, 0:1]
        y1c = data8_ref[b * 128:(b + 1) * 128, 1:2]
        x2c = data8_ref[b * 128:(b + 1) * 128, 2:3]
        y2c = data8_ref[b * 128:(b + 1) * 128, 3:4]
        ix1 = jnp.maximum(x1c, x1r)
        iy1 = jnp.maximum(y1c, y1r)
        ix2 = jnp.minimum(x2c, x2r)
        iy2 = jnp.minimum(y2c, y2r)
        inter = jnp.clip(ix2 - ix1, 0.0) * jnp.clip(iy2 - iy1, 0.0)
        area_c = (x2c - x1c) * (y2c - y1c)
        iou = inter / (area_c + area_r - inter + 1e-9)
        irow = jax.lax.broadcasted_iota(jnp.int32, (128, N2), 0) + b * 128
        jrow = jax.lax.broadcasted_iota(jnp.int32, (128, N2), 1)
        sup_ref[b * 128:(b + 1) * 128, :] = jnp.where(
            (iou > NMS_THRESH) & (jrow > irow), 1.0, 0.0)

    iota_lane = jax.lax.broadcasted_iota(jnp.int32, (1, N2), 1)
    iota128 = jax.lax.broadcasted_iota(jnp.int32, (1, 128), 1)
    keep = jnp.ones((1, N2), jnp.float32)
    for b in range(N2 // 128):
        lo = b * 128

        def blk_body(i, bk, lo=lo):
            row = sup_ref[pl.ds(lo + i, 1), :][:, lo:lo + 128]
            cur = jnp.sum(jnp.where(iota128 == i, bk, 0.0))
            return bk * (1.0 - cur * row)

        blkkeep = jax.lax.fori_loop(0, 128, blk_body, keep[:, lo:lo + 128])
        nsup = jnp.dot(blkkeep, sup_ref[lo:lo + 128, :],
                       preferred_element_type=jnp.float32)
        keep = keep * jnp.where(nsup > 0.0, 0.0, 1.0)

    valid = iota_lane < PRE_NMS
    keepv = jnp.where(valid, keep, 0.0)
    suppv = jnp.where(valid, 1.0 - keep, 0.0)
    K = jnp.sum(keepv)
    kp = keepv.reshape(16, 128)
    sp = suppv.reshape(16, 128)
    l0 = jax.lax.broadcasted_iota(jnp.int32, (128, 128), 0)
    l1 = jax.lax.broadcasted_iota(jnp.int32, (128, 128), 1)
    slt = jnp.where(l0 < l1, 1.0, 0.0)
    r0 = jax.lax.broadcasted_iota(jnp.int32, (16, 16), 0)
    r1 = jax.lax.broadcasted_iota(jnp.int32, (16, 16), 1)
    mrow = jnp.where(r1 < r0, 1.0, 0.0)
    ke = jnp.dot(kp, slt, preferred_element_type=jnp.float32) + jnp.dot(
        mrow, jnp.sum(kp, axis=1, keepdims=True),
        preferred_element_type=jnp.float32)
    se = jnp.dot(sp, slt, preferred_element_type=jnp.float32) + jnp.dot(
        mrow, jnp.sum(sp, axis=1, keepdims=True),
        preferred_element_type=jnp.float32)
    dest = jnp.where(kp > 0.5, ke, K + se)
    g16 = jax.lax.broadcasted_iota(jnp.int32, (16, 128), 0) * 128 + \
        jax.lax.broadcasted_iota(jnp.int32, (16, 128), 1)
    dest = jnp.where(g16 < PRE_NMS, dest, float(N2 - 1))
    destrow = dest.reshape(1, N2)
    dcol = jax.lax.broadcasted_iota(jnp.int32, (N2, 1), 0).astype(jnp.float32)
    onehot_t = jnp.where(dcol == destrow, 1.0, 0.0)
    out = jnp.dot(onehot_t, data8_ref[...], preferred_element_type=jnp.float32)
    lane8 = jax.lax.broadcasted_iota(jnp.int32, (N2, 8), 1)
    out = jnp.where((lane8 == 4) & (dcol >= K), -1e9, out)
    out_ref[...] = out


@functools.partial(jax.jit, static_argnames=("interpret",))
def _nms_stage(top_scores, top_boxes, interpret=False):
    data8 = jnp.concatenate([
        top_boxes, top_scores[:, None],
        jnp.zeros((PRE_NMS, 3), jnp.float32)], axis=1)
    data8 = jnp.pad(data8, ((0, N2 - PRE_NMS), (0, 0)))
    bx_row = data8[:, :4].T
    out = pl.pallas_call(
        _nms_body,
        out_shape=jax.ShapeDtypeStruct((N2, 8), jnp.float32),
        scratch_shapes=[pltpu.VMEM((N2, N2), jnp.float32)],
        interpret=interpret,
    )(data8, bx_row)
    return out[:POST_NMS, :5]


@jax.jit
def kernel(features, W_conv, b_conv, W_logits, b_logits, W_deltas, b_deltas):
    scores, boxes = _dense_stage(features, W_conv, b_conv, W_logits,
                                 b_logits, W_deltas, b_deltas)
    top_scores, idx = jax.lax.top_k(scores, PRE_NMS)
    top_boxes = boxes[idx]
    return _nms_stage(top_scores, top_boxes)
